# Initial kernel scaffold; baseline (speedup 1.0000x reference)
import jax
import jax.numpy as jnp
from jax.experimental import pallas as pl


def kernel(item_ids, score_embedding):
    raise NotImplementedError("write your pallas kernel here")


# speedup vs baseline: 98.7282x; 98.7282x over previous
import functools

import jax
import jax.numpy as jnp
from jax import lax
from jax.experimental import pallas as pl
from jax.experimental.pallas import tpu as pltpu
from jax.experimental.pallas import tpu_sc as plsc

NUM_ITEMS = 1000000
BATCH = 16384
HIST = 200
N = BATCH * HIST

NC = 2
NS = 16
NW = NC * NS

N_PER_W = N // NW
ROW = 128
ROWS_PER_CHUNK = 16
CHUNK = ROW * ROWS_PER_CHUNK
N_CHUNKS = N_PER_W // CHUNK


def _gather_kernel(table_hbm, idx_hbm, out_hbm, idx_v, rows_v, sem):
    wid = lax.axis_index("s") * NC + lax.axis_index("c")
    base = wid * N_PER_W

    def body(i, carry):
        off = pl.multiple_of(base + i * CHUNK, CHUNK)
        pltpu.sync_copy(idx_hbm.at[pl.ds(off, CHUNK)], idx_v)
        copies = [
            pltpu.async_copy(
                table_hbm.at[idx_v.at[pl.ds(j * ROW, ROW)]],
                rows_v.at[pl.ds(j * ROW, ROW)],
                sem,
            )
            for j in range(ROWS_PER_CHUNK)
        ]
        for c in copies:
            c.wait()
        pltpu.sync_copy(rows_v, out_hbm.at[pl.ds(off, CHUNK)])
        return carry

    lax.fori_loop(0, N_CHUNKS, body, 0)


@jax.jit
def kernel(item_ids, score_embedding):
    idx = item_ids.reshape(N)
    table = score_embedding.reshape(NUM_ITEMS)
    mesh = plsc.VectorSubcoreMesh(core_axis_name="c", subcore_axis_name="s")
    out = pl.kernel(
        _gather_kernel,
        mesh=mesh,
        out_type=jax.ShapeDtypeStruct((N,), jnp.float32),
        scratch_types=[
            pltpu.VMEM((CHUNK,), jnp.int32),
            pltpu.VMEM((CHUNK,), jnp.float32),
            pltpu.SemaphoreType.DMA,
        ],
    )(table, idx)
    return out.reshape(BATCH, HIST, 1)


# speedup vs baseline: 120.6834x; 1.2224x over previous
import functools

import jax
import jax.numpy as jnp
from jax import lax
from jax.experimental import pallas as pl
from jax.experimental.pallas import tpu as pltpu
from jax.experimental.pallas import tpu_sc as plsc

NUM_ITEMS = 1000000
BATCH = 16384
HIST = 200
N = BATCH * HIST

NC = 2
NS = 16
NW = NC * NS

N_PER_W = N // NW
CHUNK = 12800
N_CHUNKS = N_PER_W // CHUNK


def _gather_kernel(table_hbm, idx_hbm, out_hbm, idx_v, rows_v, sem):
    wid = lax.axis_index("s") * NC + lax.axis_index("c")
    base = wid * N_PER_W

    def body(i, carry):
        off = pl.multiple_of(base + i * CHUNK, CHUNK)
        pltpu.sync_copy(idx_hbm.at[pl.ds(off, CHUNK)], idx_v)
        pltpu.async_copy(table_hbm.at[idx_v], rows_v, sem).wait()
        pltpu.sync_copy(rows_v, out_hbm.at[pl.ds(off, CHUNK)])
        return carry

    lax.fori_loop(0, N_CHUNKS, body, 0)


@jax.jit
def kernel(item_ids, score_embedding):
    idx = item_ids.reshape(N)
    table = score_embedding.reshape(NUM_ITEMS)
    mesh = plsc.VectorSubcoreMesh(core_axis_name="c", subcore_axis_name="s")
    out = pl.kernel(
        _gather_kernel,
        mesh=mesh,
        out_type=jax.ShapeDtypeStruct((N,), jnp.float32),
        scratch_types=[
            pltpu.VMEM((CHUNK,), jnp.int32),
            pltpu.VMEM((CHUNK,), jnp.float32),
            pltpu.SemaphoreType.DMA,
        ],
    )(table, idx)
    return out.reshape(BATCH, HIST, 1)


# speedup vs baseline: 173.8827x; 1.4408x over previous
import functools

import jax
import jax.numpy as jnp
from jax import lax
from jax.experimental import pallas as pl
from jax.experimental.pallas import tpu as pltpu
from jax.experimental.pallas import tpu_sc as plsc

NUM_ITEMS = 1000000
BATCH = 16384
HIST = 200
N = BATCH * HIST

NC = 2
NS = 16
NW = NC * NS

N_PER_W = N // NW
CHUNK = 12800
N_CHUNKS = N_PER_W // CHUNK
STAGE_PIECE = 10000
N_PIECES = NUM_ITEMS // STAGE_PIECE


def _gather_kernel(table_hbm, idx_hbm, out_hbm, tab_s, idx_v, rows_v, sem):
    cid = lax.axis_index("c")
    sid = lax.axis_index("s")
    wid = sid * NC + cid
    base = wid * N_PER_W

    def sbody(p, carry):
        @pl.when(lax.rem(p, NS) == sid)
        def _():
            off = pl.multiple_of(p * STAGE_PIECE, 8)
            pltpu.sync_copy(table_hbm.at[pl.ds(off, STAGE_PIECE)],
                            rows_v.at[pl.ds(0, STAGE_PIECE)])
            pltpu.sync_copy(rows_v.at[pl.ds(0, STAGE_PIECE)],
                            tab_s.at[pl.ds(off, STAGE_PIECE)])
        return carry

    lax.fori_loop(0, N_PIECES, sbody, 0)
    plsc.subcore_barrier()

    def body(i, carry):
        off = pl.multiple_of(base + i * CHUNK, CHUNK)
        pltpu.sync_copy(idx_hbm.at[pl.ds(off, CHUNK)], idx_v)
        pltpu.async_copy(tab_s.at[idx_v], rows_v, sem).wait()
        pltpu.sync_copy(rows_v, out_hbm.at[pl.ds(off, CHUNK)])
        return carry

    lax.fori_loop(0, N_CHUNKS, body, 0)


@jax.jit
def kernel(item_ids, score_embedding):
    idx = item_ids.reshape(N)
    table = score_embedding.reshape(NUM_ITEMS)
    mesh = plsc.VectorSubcoreMesh(core_axis_name="c", subcore_axis_name="s")
    out = pl.kernel(
        _gather_kernel,
        mesh=mesh,
        out_type=jax.ShapeDtypeStruct((N,), jnp.float32),
        scratch_types=[
            pltpu.VMEM_SHARED((NUM_ITEMS,), jnp.float32),
            pltpu.VMEM((CHUNK,), jnp.int32),
            pltpu.VMEM((CHUNK,), jnp.float32),
            pltpu.SemaphoreType.DMA,
        ],
    )(table, idx)
    return out.reshape(BATCH, HIST, 1)


# speedup vs baseline: 186.7524x; 1.0740x over previous
import functools

import jax
import jax.numpy as jnp
from jax import lax
from jax.experimental import pallas as pl
from jax.experimental.pallas import tpu as pltpu
from jax.experimental.pallas import tpu_sc as plsc

NUM_ITEMS = 1000000
BATCH = 16384
HIST = 200
N = BATCH * HIST

NC = 2
NS = 16
NW = NC * NS

N_PER_W = N // NW
CHUNK = 12800
N_CHUNKS = N_PER_W // CHUNK
STAGE_PIECE = 10000
N_PIECES = NUM_ITEMS // STAGE_PIECE


def _gather_kernel(table_hbm, idx_hbm, out_hbm, tab_s,
                   idx0, idx1, rows0, rows1, gsem, ssem):
    cid = lax.axis_index("c")
    sid = lax.axis_index("s")
    wid = sid * NC + cid
    base = wid * N_PER_W

    def coff(c):
        return pl.multiple_of(base + c * CHUNK, CHUNK)

    def sbody(p, carry):
        @pl.when(lax.rem(p, NS) == sid)
        def _():
            off = pl.multiple_of(p * STAGE_PIECE, 8)
            pltpu.sync_copy(table_hbm.at[pl.ds(off, STAGE_PIECE)],
                            rows0.at[pl.ds(0, STAGE_PIECE)])
            pltpu.sync_copy(rows0.at[pl.ds(0, STAGE_PIECE)],
                            tab_s.at[pl.ds(off, STAGE_PIECE)])
        return carry

    lax.fori_loop(0, N_PIECES, sbody, 0)

    idx = (idx0, idx1)
    rows = (rows0, rows1)
    pltpu.sync_copy(idx_hbm.at[pl.ds(coff(0), CHUNK)], idx[0])
    pltpu.sync_copy(idx_hbm.at[pl.ds(coff(1), CHUNK)], idx[1])
    plsc.subcore_barrier()

    gs = [pltpu.async_copy(tab_s.at[idx[0]], rows[0], gsem),
          pltpu.async_copy(tab_s.at[idx[1]], rows[1], gsem)]
    for c in range(N_CHUNKS):
        b = c & 1
        gs[b].wait()
        s = pltpu.async_copy(rows[b], out_hbm.at[pl.ds(coff(c), CHUNK)], ssem)
        if c + 2 < N_CHUNKS:
            pltpu.sync_copy(idx_hbm.at[pl.ds(coff(c + 2), CHUNK)], idx[b])
            s.wait()
            gs[b] = pltpu.async_copy(tab_s.at[idx[b]], rows[b], gsem)
        else:
            s.wait()


@jax.jit
def kernel(item_ids, score_embedding):
    idx = item_ids.reshape(N)
    table = score_embedding.reshape(NUM_ITEMS)
    mesh = plsc.VectorSubcoreMesh(core_axis_name="c", subcore_axis_name="s")
    out = pl.kernel(
        _gather_kernel,
        mesh=mesh,
        out_type=jax.ShapeDtypeStruct((N,), jnp.float32),
        scratch_types=[
            pltpu.VMEM_SHARED((NUM_ITEMS,), jnp.float32),
            pltpu.VMEM((CHUNK,), jnp.int32),
            pltpu.VMEM((CHUNK,), jnp.int32),
            pltpu.VMEM((CHUNK,), jnp.float32),
            pltpu.VMEM((CHUNK,), jnp.float32),
            pltpu.SemaphoreType.DMA,
            pltpu.SemaphoreType.DMA,
        ],
    )(table, idx)
    return out.reshape(BATCH, HIST, 1)
